# baseline (speedup 1.0000x reference)
import functools

import jax
import jax.numpy as jnp
from jax import lax
from jax.experimental import pallas as pl
from jax.experimental.pallas import tpu as pltpu
from jax.experimental.pallas import tpu_sc as plsc

D = 64
B = 16384
V = 1000000
NC, NS, L = 2, 16, 16
NW = NC * NS
BPW = B // NW
NBLK = BPW // L
NCOLS = (V + 127) // 128
NCP = (NCOLS + NW - 1) // NW
RB = 16


def _pass(list_hbm, tbl_hbm, stage_hbm, idx_v, cb_idx, cb_b, colbuf,
          rowbuf, sem_f, sem_w, c0, c1, lane):
    pltpu.sync_copy(list_hbm, idx_v)
    lo = c0 * 128
    hi = c1 * 128

    def compact_body(i, off):
        v = idx_v[pl.ds(i * L, L)]
        m = (v >= lo) & (v < hi)
        plsc.store_compressed(cb_idx.at[pl.ds(off, L)], v, mask=m)
        plsc.store_compressed(cb_b.at[pl.ds(off, L)], i * L + lane, mask=m)
        return off + plsc.all_reduce_population_count(m)[0]

    cnt = lax.fori_loop(0, B // L, compact_body, 0)
    nch = (cnt + L - 1) // L

    def fetch(ci, par):
        c = c0 + ci
        @pl.when((c < NCOLS) & (ci < NCP))
        def _():
            off = pl.multiple_of(c * 128, 128)
            pltpu.async_copy(tbl_hbm.at[:, pl.ds(off, 128)],
                             colbuf.at[par], sem_f)

    fetch(0, 0)

    def col_body(ci, mc):
        par = ci % 2
        c = c0 + ci

        @pl.when(c < NCOLS)
        def _():
            pltpu.make_async_copy(tbl_hbm.at[:, pl.ds(0, 128)],
                                  colbuf.at[par], sem_f).wait()
        fetch(ci + 1, 1 - par)

        def chunk_body(ch, mc):
            p0 = ch * L
            v = cb_idx[pl.ds(p0, L)]
            m = ((v >> 7) == c) & ((p0 + lane) < cnt)
            n = plsc.all_reduce_population_count(m)[0]

            def match_body(_, carry):
                m, mc = carry
                k = plsc.all_reduce_ffs(m)[0]
                iv = cb_idx[pl.ds(p0 + k, L)][0]
                b = cb_b[pl.ds(p0 + k, L)][0]
                i_loc = iv & 127
                slot = mc % RB

                @pl.when(mc >= RB)
                def _():
                    pltpu.make_async_copy(
                        rowbuf.at[pl.ds(0, D)],
                        stage_hbm.at[pl.ds(0, D)], sem_w).wait()

                for g in range(D // L):
                    val = plsc.load_gather(
                        colbuf,
                        [jnp.full((L,), par, jnp.int32), g * L + lane,
                         jnp.full((L,), i_loc, jnp.int32)])
                    rowbuf[pl.ds(slot * D + g * L, L)] = val
                pltpu.async_copy(rowbuf.at[pl.ds(slot * D, D)],
                                 stage_hbm.at[pl.ds(b * D, D)], sem_w)
                return m & (lane != k), mc + 1

            _, mc = lax.fori_loop(0, n, match_body, (m, mc))
            return mc

        ncc = jnp.where(c < NCOLS, nch, 0)
        return lax.fori_loop(0, ncc, chunk_body, mc)

    mc = lax.fori_loop(0, NCP, col_body, 0)

    def fin_drain(_, carry):
        pltpu.make_async_copy(rowbuf.at[pl.ds(0, D)],
                              stage_hbm.at[pl.ds(0, D)], sem_w).wait()
        return carry

    lax.fori_loop(0, jnp.minimum(mc, RB), fin_drain, 0)


def _scan_body(user_hbm, item_i_hbm, item_j_hbm, eut_hbm, eit_hbm,
               su_hbm, si_hbm, sj_hbm,
               idx_v, cb_idx, cb_b, colbuf, rowbuf, sem_f, sem_w):
    wid = lax.axis_index("s") * NC + lax.axis_index("c")
    c0 = wid * NCP
    c1 = c0 + NCP
    lane = lax.iota(jnp.int32, L)
    _pass(user_hbm, eut_hbm, su_hbm, idx_v, cb_idx, cb_b, colbuf,
          rowbuf, sem_f, sem_w, c0, c1, lane)
    _pass(item_i_hbm, eit_hbm, si_hbm, idx_v, cb_idx, cb_b, colbuf,
          rowbuf, sem_f, sem_w, c0, c1, lane)
    _pass(item_j_hbm, eit_hbm, sj_hbm, idx_v, cb_idx, cb_b, colbuf,
          rowbuf, sem_f, sem_w, c0, c1, lane)


def _dot_body(su_hbm, si_hbm, sj_hbm, out_i_hbm, out_j_hbm,
              u_v, i_v, j_v, pi_v, pj_v):
    wid = lax.axis_index("s") * NC + lax.axis_index("c")
    base = wid * BPW
    pltpu.sync_copy(su_hbm.at[pl.ds(base * D, BPW * D)], u_v)
    pltpu.sync_copy(si_hbm.at[pl.ds(base * D, BPW * D)], i_v)
    pltpu.sync_copy(sj_hbm.at[pl.ds(base * D, BPW * D)], j_v)
    lane = lax.iota(jnp.int32, L)

    def blk_body(blk, carry):
        rbase = (blk * L + lane) * D
        acc_i = jnp.zeros((L,), jnp.float32)
        acc_j = jnp.zeros((L,), jnp.float32)
        for f in range(D):
            u = plsc.load_gather(u_v, [rbase + f])
            a = plsc.load_gather(i_v, [rbase + f])
            b = plsc.load_gather(j_v, [rbase + f])
            acc_i = acc_i + u * a
            acc_j = acc_j + u * b
        pi_v[pl.ds(blk * L, L)] = acc_i
        pj_v[pl.ds(blk * L, L)] = acc_j
        return carry

    lax.fori_loop(0, NBLK, blk_body, 0)
    pltpu.sync_copy(pi_v, out_i_hbm.at[pl.ds(base, BPW)])
    pltpu.sync_copy(pj_v, out_j_hbm.at[pl.ds(base, BPW)])


@jax.jit
def _run(user, item_i, item_j, embed_user, embed_item):
    mesh = plsc.VectorSubcoreMesh(core_axis_name="c", subcore_axis_name="s")
    params = pltpu.CompilerParams(
        needs_layout_passes=False, use_tc_tiling_on_sc=True)
    scan = pl.kernel(
        _scan_body,
        out_type=(
            jax.ShapeDtypeStruct((B * D,), jnp.float32),
            jax.ShapeDtypeStruct((B * D,), jnp.float32),
            jax.ShapeDtypeStruct((B * D,), jnp.float32),
        ),
        mesh=mesh,
        scratch_types=[
            pltpu.VMEM((B,), jnp.int32),
            pltpu.VMEM((B + L,), jnp.int32),
            pltpu.VMEM((B + L,), jnp.int32),
            pltpu.VMEM((2, D, 128), jnp.float32),
            pltpu.VMEM((RB * D,), jnp.float32),
            pltpu.SemaphoreType.DMA,
            pltpu.SemaphoreType.DMA,
        ],
        compiler_params=params,
    )
    su, si, sj = scan(user, item_i, item_j, embed_user.T, embed_item.T)
    dot = pl.kernel(
        _dot_body,
        out_type=(
            jax.ShapeDtypeStruct((B,), jnp.float32),
            jax.ShapeDtypeStruct((B,), jnp.float32),
        ),
        mesh=mesh,
        scratch_types=[
            pltpu.VMEM((BPW * D,), jnp.float32),
            pltpu.VMEM((BPW * D,), jnp.float32),
            pltpu.VMEM((BPW * D,), jnp.float32),
            pltpu.VMEM((BPW,), jnp.float32),
            pltpu.VMEM((BPW,), jnp.float32),
        ],
        compiler_params=params,
    )
    return dot(su, si, sj)


def kernel(user, item_i, item_j, embed_user, embed_item):
    return _run(user.astype(jnp.int32), item_i.astype(jnp.int32),
                item_j.astype(jnp.int32), embed_user, embed_item)


# speedup vs baseline: 1.0505x; 1.0505x over previous
import functools

import jax
import jax.numpy as jnp
from jax import lax
from jax.experimental import pallas as pl
from jax.experimental.pallas import tpu as pltpu
from jax.experimental.pallas import tpu_sc as plsc

D = 64
B = 16384
V = 1000000
NC, NS, L = 2, 16, 16
NW = NC * NS
BPW = B // NW
NBLK = BPW // L
NCOLS = (V + 127) // 128
NCP = (NCOLS + NW - 1) // NW
RB = 16


NBUF = 4


def _compact(list_hbm, idx_v, cb, lo, hi, lane):
    pltpu.sync_copy(list_hbm, idx_v)

    def compact_body(i, off):
        v = idx_v[pl.ds(i * L, L)]
        m = (v >= lo) & (v < hi)
        code = ((v - lo) << 14) | (i * L + lane)
        plsc.store_compressed(cb.at[pl.ds(off, L)], code, mask=m)
        return off + plsc.all_reduce_population_count(m)[0]

    return lax.fori_loop(0, B // L, compact_body, 0)


def _scan_list(cb, cnt, ci, par, colbuf, rowbuf, stage_hbm, sem_w,
               lane, mc):
    nch = (cnt + L - 1) // L

    def chunk_body(ch, mc):
        p0 = ch * L
        v = cb[pl.ds(p0, L)]
        m = ((v >> 21) == ci) & ((p0 + lane) < cnt)
        n = plsc.all_reduce_population_count(m)[0]

        def match_body(_, carry):
            m, mc = carry
            k = plsc.all_reduce_ffs(m)[0]
            code = cb[pl.ds(p0 + k, L)][0]
            i_loc = (code >> 14) & 127
            b = code & 16383
            slot = mc % RB

            @pl.when(mc >= RB)
            def _():
                pltpu.make_async_copy(
                    rowbuf.at[pl.ds(0, D)],
                    stage_hbm.at[pl.ds(0, D)], sem_w).wait()

            for g in range(D // L):
                val = plsc.load_gather(
                    colbuf,
                    [jnp.full((L,), par, jnp.int32), g * L + lane,
                     jnp.full((L,), i_loc, jnp.int32)])
                rowbuf[pl.ds(slot * D + g * L, L)] = val
            pltpu.async_copy(rowbuf.at[pl.ds(slot * D, D)],
                             stage_hbm.at[pl.ds(b * D, D)], sem_w)
            return m & (lane != k), mc + 1

        _, mc = lax.fori_loop(0, n, match_body, (m, mc))
        return mc

    return lax.fori_loop(0, nch, chunk_body, mc)


def _col_loop(tbl_hbm, colbuf, sem_f, c0, process):
    def fetch(ci, par):
        c = c0 + ci
        @pl.when((c < NCOLS) & (ci < NCP))
        def _():
            off = pl.multiple_of(c * 128, 128)
            pltpu.async_copy(tbl_hbm.at[:, pl.ds(off, 128)],
                             colbuf.at[par], sem_f)

    for w in range(NBUF - 1):
        fetch(w, w)

    def col_body(ci, mc):
        par = ci % NBUF
        c = c0 + ci

        @pl.when(c < NCOLS)
        def _():
            pltpu.make_async_copy(tbl_hbm.at[:, pl.ds(0, 128)],
                                  colbuf.at[par], sem_f).wait()
        fetch(ci + NBUF - 1, (ci + NBUF - 1) % NBUF)
        return jnp.where(c < NCOLS, process(ci, par, mc), mc)

    return lax.fori_loop(0, NCP, col_body, 0)


def _fin_drain(rowbuf, stage_hbm, sem_w, mc):
    def fin(_, carry):
        pltpu.make_async_copy(rowbuf.at[pl.ds(0, D)],
                              stage_hbm.at[pl.ds(0, D)], sem_w).wait()
        return carry

    lax.fori_loop(0, jnp.minimum(mc, RB), fin, 0)


def _scan_body(user_hbm, item_i_hbm, item_j_hbm, eut_hbm, eit_hbm,
               su_hbm, si_hbm, sj_hbm,
               idx_v, cb_u, cb_j, colbuf, rowbuf, sem_f, sem_w):
    wid = lax.axis_index("s") * NC + lax.axis_index("c")
    c0 = wid * NCP
    lo = c0 * 128
    hi = lo + NCP * 128
    lane = lax.iota(jnp.int32, L)

    cnt_u = _compact(user_hbm, idx_v, cb_u, lo, hi, lane)

    def proc_u(ci, par, mc):
        return _scan_list(cb_u, cnt_u, ci, par, colbuf, rowbuf,
                          su_hbm, sem_w, lane, mc)

    mc = _col_loop(eut_hbm, colbuf, sem_f, c0, proc_u)
    _fin_drain(rowbuf, su_hbm, sem_w, mc)

    cnt_i = _compact(item_i_hbm, idx_v, cb_u, lo, hi, lane)
    cnt_j = _compact(item_j_hbm, idx_v, cb_j, lo, hi, lane)

    def proc_ij(ci, par, mc):
        mc = _scan_list(cb_u, cnt_i, ci, par, colbuf, rowbuf,
                        si_hbm, sem_w, lane, mc)
        return _scan_list(cb_j, cnt_j, ci, par, colbuf, rowbuf,
                          sj_hbm, sem_w, lane, mc)

    mc = _col_loop(eit_hbm, colbuf, sem_f, c0, proc_ij)
    _fin_drain(rowbuf, sj_hbm, sem_w, mc)


def _dot_body(su_hbm, si_hbm, sj_hbm, out_i_hbm, out_j_hbm,
              u_v, i_v, j_v, pi_v, pj_v):
    wid = lax.axis_index("s") * NC + lax.axis_index("c")
    base = wid * BPW
    pltpu.sync_copy(su_hbm.at[pl.ds(base * D, BPW * D)], u_v)
    pltpu.sync_copy(si_hbm.at[pl.ds(base * D, BPW * D)], i_v)
    pltpu.sync_copy(sj_hbm.at[pl.ds(base * D, BPW * D)], j_v)
    lane = lax.iota(jnp.int32, L)

    def blk_body(blk, carry):
        rbase = (blk * L + lane) * D
        acc_i = jnp.zeros((L,), jnp.float32)
        acc_j = jnp.zeros((L,), jnp.float32)
        for f in range(D):
            u = plsc.load_gather(u_v, [rbase + f])
            a = plsc.load_gather(i_v, [rbase + f])
            b = plsc.load_gather(j_v, [rbase + f])
            acc_i = acc_i + u * a
            acc_j = acc_j + u * b
        pi_v[pl.ds(blk * L, L)] = acc_i
        pj_v[pl.ds(blk * L, L)] = acc_j
        return carry

    lax.fori_loop(0, NBLK, blk_body, 0)
    pltpu.sync_copy(pi_v, out_i_hbm.at[pl.ds(base, BPW)])
    pltpu.sync_copy(pj_v, out_j_hbm.at[pl.ds(base, BPW)])


@jax.jit
def _run(user, item_i, item_j, embed_user, embed_item):
    mesh = plsc.VectorSubcoreMesh(core_axis_name="c", subcore_axis_name="s")
    params = pltpu.CompilerParams(
        needs_layout_passes=False, use_tc_tiling_on_sc=True)
    scan = pl.kernel(
        _scan_body,
        out_type=(
            jax.ShapeDtypeStruct((B * D,), jnp.float32),
            jax.ShapeDtypeStruct((B * D,), jnp.float32),
            jax.ShapeDtypeStruct((B * D,), jnp.float32),
        ),
        mesh=mesh,
        scratch_types=[
            pltpu.VMEM((B,), jnp.int32),
            pltpu.VMEM((B + L,), jnp.int32),
            pltpu.VMEM((B + L,), jnp.int32),
            pltpu.VMEM((NBUF, D, 128), jnp.float32),
            pltpu.VMEM((RB * D,), jnp.float32),
            pltpu.SemaphoreType.DMA,
            pltpu.SemaphoreType.DMA,
        ],
        compiler_params=params,
    )
    su, si, sj = scan(user, item_i, item_j, embed_user.T, embed_item.T)
    dot = pl.kernel(
        _dot_body,
        out_type=(
            jax.ShapeDtypeStruct((B,), jnp.float32),
            jax.ShapeDtypeStruct((B,), jnp.float32),
        ),
        mesh=mesh,
        scratch_types=[
            pltpu.VMEM((BPW * D,), jnp.float32),
            pltpu.VMEM((BPW * D,), jnp.float32),
            pltpu.VMEM((BPW * D,), jnp.float32),
            pltpu.VMEM((BPW,), jnp.float32),
            pltpu.VMEM((BPW,), jnp.float32),
        ],
        compiler_params=params,
    )
    return dot(su, si, sj)


def kernel(user, item_i, item_j, embed_user, embed_item):
    return _run(user.astype(jnp.int32), item_i.astype(jnp.int32),
                item_j.astype(jnp.int32), embed_user, embed_item)


# speedup vs baseline: 1.1366x; 1.0819x over previous
import functools

import jax
import jax.numpy as jnp
from jax import lax
from jax.experimental import pallas as pl
from jax.experimental.pallas import tpu as pltpu
from jax.experimental.pallas import tpu_sc as plsc

D = 64
B = 16384
NC, NS, L = 2, 16, 16
NW = NC * NS
BPW = B // NW
NBLK = BPW // L


def _body(user_hbm, item_i_hbm, item_j_hbm, eu_hbm, ei_hbm,
          out_i_hbm, out_j_hbm,
          uidx_v, iidx_v, jidx_v, eu_r, ei_r, ej_r, pi_v, pj_v, sem):
    wid = lax.axis_index("s") * NC + lax.axis_index("c")
    base = wid * BPW

    pltpu.sync_copy(user_hbm.at[pl.ds(base, BPW)], uidx_v)
    pltpu.sync_copy(item_i_hbm.at[pl.ds(base, BPW)], iidx_v)
    pltpu.sync_copy(item_j_hbm.at[pl.ds(base, BPW)], jidx_v)

    lane = lax.iota(jnp.int32, L)

    def blk_body(blk, carry):
        sl = pl.ds(blk * L, L)
        uu = uidx_v[sl]
        iiv = iidx_v[sl]
        jjv = jidx_v[sl]
        handles = []
        for k in range(L):
            iu = uu[k]
            ii = iiv[k]
            ij = jjv[k]
            dk = pl.ds(k, 1)
            handles.append(pltpu.async_copy(eu_hbm.at[pl.ds(iu, 1)], eu_r.at[dk], sem))
            handles.append(pltpu.async_copy(ei_hbm.at[pl.ds(ii, 1)], ei_r.at[dk], sem))
            handles.append(pltpu.async_copy(ei_hbm.at[pl.ds(ij, 1)], ej_r.at[dk], sem))
        for h in handles:
            h.wait()
        acc_i = jnp.zeros((L,), jnp.float32)
        acc_j = jnp.zeros((L,), jnp.float32)
        for f in range(D):
            cols = jnp.full((L,), f, jnp.int32)
            u = plsc.load_gather(eu_r, [lane, cols])
            a = plsc.load_gather(ei_r, [lane, cols])
            b = plsc.load_gather(ej_r, [lane, cols])
            acc_i = acc_i + u * a
            acc_j = acc_j + u * b
        pi_v[sl] = acc_i
        pj_v[sl] = acc_j
        return carry

    lax.fori_loop(0, NBLK, blk_body, 0)

    pltpu.sync_copy(pi_v, out_i_hbm.at[pl.ds(base, BPW)])
    pltpu.sync_copy(pj_v, out_j_hbm.at[pl.ds(base, BPW)])


@jax.jit
def _run(user, item_i, item_j, embed_user, embed_item):
    mesh = plsc.VectorSubcoreMesh(core_axis_name="c", subcore_axis_name="s")
    k = pl.kernel(
        _body,
        out_type=(
            jax.ShapeDtypeStruct((B,), jnp.float32),
            jax.ShapeDtypeStruct((B,), jnp.float32),
        ),
        mesh=mesh,
        scratch_types=[
            pltpu.VMEM((BPW,), jnp.int32),
            pltpu.VMEM((BPW,), jnp.int32),
            pltpu.VMEM((BPW,), jnp.int32),
            pltpu.VMEM((L, D), jnp.float32),
            pltpu.VMEM((L, D), jnp.float32),
            pltpu.VMEM((L, D), jnp.float32),
            pltpu.VMEM((BPW,), jnp.float32),
            pltpu.VMEM((BPW,), jnp.float32),
            pltpu.SemaphoreType.DMA,
        ],
        compiler_params=pltpu.CompilerParams(
            needs_layout_passes=False, use_tc_tiling_on_sc=True),
    )
    return k(user, item_i, item_j, embed_user, embed_item)


def kernel(user, item_i, item_j, embed_user, embed_item):
    return _run(user.astype(jnp.int32), item_i.astype(jnp.int32),
                item_j.astype(jnp.int32), embed_user, embed_item)


# speedup vs baseline: 2.2774x; 2.0038x over previous
import functools

import jax
import jax.numpy as jnp
from jax import lax
from jax.experimental import pallas as pl
from jax.experimental.pallas import tpu as pltpu
from jax.experimental.pallas import tpu_sc as plsc

D = 64
B = 16384
V = 1000000
NC, NS, L = 2, 16, 16
NW = NC * NS
BPW = B // NW
NBLK = BPW // L
NCOLS = (V + 127) // 128
NCP = (NCOLS + NW - 1) // NW
RB = 16


NBUF = 4
NG = 8
GW = 32


def _compact(list_hbm, idx_v, cb, lo, hi, lane):
    pltpu.sync_copy(list_hbm, idx_v.at[pl.ds(0, B)])

    def compact_body(i, off):
        v = idx_v[pl.ds(i * L, L)]
        m = (v >= lo) & (v < hi)
        code = ((v - lo) << 14) | (i * L + lane)
        plsc.store_compressed(cb.at[pl.ds(off, L)], code, mask=m)
        return off + plsc.all_reduce_population_count(m)[0]

    return lax.fori_loop(0, B // L, compact_body, 0)


def _partition(cb, cnt, dst, lane):
    nch = (cnt + L - 1) // L
    bounds = [0]
    cur = 0
    for g in range(NG):
        def cnt_body(ch, acc, g=g):
            p0 = ch * L
            v = cb[pl.ds(p0, L)]
            m = ((v >> 26) == g) & ((p0 + lane) < cnt)
            return acc + plsc.all_reduce_population_count(m)[0]

        cur = cur + lax.fori_loop(0, nch, cnt_body, 0)
        bounds.append(cur)
    for g in range(NG):
        def part_body(ch, off, g=g):
            p0 = ch * L
            v = cb[pl.ds(p0, L)]
            m = ((v >> 26) == g) & ((p0 + lane) < cnt)
            plsc.store_compressed(dst.at[pl.ds(off, L)], v, mask=m)
            return off + plsc.all_reduce_population_count(m)[0]

        lax.fori_loop(0, nch, part_body, bounds[g])
    return bounds


def _bounds_at(bounds, g):
    lo = bounds[0]
    hi = bounds[1]
    for i in range(1, NG):
        lo = jnp.where(g >= i, bounds[i], lo)
        hi = jnp.where(g >= i, bounds[i + 1], hi)
    return lo, hi


def _scan_list(cb, bounds, ci, par, colbuf, rowbuf, stage_hbm, sem_w,
               lane, mc):
    start, end = _bounds_at(bounds, ci >> 5)
    nch = (end - start + L - 1) // L

    def chunk_body(ch, mc):
        p0 = start + ch * L
        v = cb[pl.ds(p0, L)]
        m = ((v >> 21) == ci) & ((p0 + lane) < end)
        n = plsc.all_reduce_population_count(m)[0]

        def match_body(_, carry):
            m, mc = carry
            k = plsc.all_reduce_ffs(m)[0]
            code = cb[pl.ds(p0 + k, L)][0]
            i_loc = (code >> 14) & 127
            b = code & 16383
            slot = mc % RB

            @pl.when(mc >= RB)
            def _():
                pltpu.make_async_copy(
                    rowbuf.at[pl.ds(0, D)],
                    stage_hbm.at[pl.ds(0, D)], sem_w).wait()

            for g in range(D // L):
                val = plsc.load_gather(
                    colbuf,
                    [jnp.full((L,), par, jnp.int32), g * L + lane,
                     jnp.full((L,), i_loc, jnp.int32)])
                rowbuf[pl.ds(slot * D + g * L, L)] = val
            pltpu.async_copy(rowbuf.at[pl.ds(slot * D, D)],
                             stage_hbm.at[pl.ds(b * D, D)], sem_w)
            return m & (lane != k), mc + 1

        _, mc = lax.fori_loop(0, n, match_body, (m, mc))
        return mc

    return lax.fori_loop(0, nch, chunk_body, mc)


def _col_loop(tbl_hbm, colbuf, sem_f, c0, process):
    def fetch(ci, par):
        c = c0 + ci
        @pl.when((c < NCOLS) & (ci < NCP))
        def _():
            off = pl.multiple_of(c * 128, 128)
            pltpu.async_copy(tbl_hbm.at[:, pl.ds(off, 128)],
                             colbuf.at[par], sem_f)

    for w in range(NBUF - 1):
        fetch(w, w)

    def col_body(ci, mc):
        par = ci % NBUF
        c = c0 + ci

        @pl.when(c < NCOLS)
        def _():
            pltpu.make_async_copy(tbl_hbm.at[:, pl.ds(0, 128)],
                                  colbuf.at[par], sem_f).wait()
        fetch(ci + NBUF - 1, (ci + NBUF - 1) % NBUF)
        return jnp.where(c < NCOLS, process(ci, par, mc), mc)

    return lax.fori_loop(0, NCP, col_body, 0)


def _fin_drain(rowbuf, stage_hbm, sem_w, mc):
    def fin(_, carry):
        pltpu.make_async_copy(rowbuf.at[pl.ds(0, D)],
                              stage_hbm.at[pl.ds(0, D)], sem_w).wait()
        return carry

    lax.fori_loop(0, jnp.minimum(mc, RB), fin, 0)


def _scan_body(user_hbm, item_i_hbm, item_j_hbm, eut_hbm, eit_hbm,
               su_hbm, si_hbm, sj_hbm,
               idx_v, cb_u, cb_j, colbuf, rowbuf, sem_f, sem_w):
    wid = lax.axis_index("s") * NC + lax.axis_index("c")
    c0 = wid * NCP
    lo = c0 * 128
    hi = lo + NCP * 128
    lane = lax.iota(jnp.int32, L)

    cnt_u = _compact(user_hbm, idx_v, cb_u, lo, hi, lane)
    bnd_u = _partition(cb_u, cnt_u, cb_j, lane)

    def proc_u(ci, par, mc):
        return _scan_list(cb_j, bnd_u, ci, par, colbuf, rowbuf,
                          su_hbm, sem_w, lane, mc)

    mc = _col_loop(eut_hbm, colbuf, sem_f, c0, proc_u)
    _fin_drain(rowbuf, su_hbm, sem_w, mc)

    cnt_i = _compact(item_i_hbm, idx_v, cb_u, lo, hi, lane)
    cnt_j = _compact(item_j_hbm, idx_v, cb_j, lo, hi, lane)
    bnd_i = _partition(cb_u, cnt_i, idx_v, lane)
    bnd_j = _partition(cb_j, cnt_j, cb_u, lane)

    def proc_ij(ci, par, mc):
        mc = _scan_list(idx_v, bnd_i, ci, par, colbuf, rowbuf,
                        si_hbm, sem_w, lane, mc)
        return _scan_list(cb_u, bnd_j, ci, par, colbuf, rowbuf,
                          sj_hbm, sem_w, lane, mc)

    mc = _col_loop(eit_hbm, colbuf, sem_f, c0, proc_ij)
    _fin_drain(rowbuf, sj_hbm, sem_w, mc)


def _dot_body(su_hbm, si_hbm, sj_hbm, out_i_hbm, out_j_hbm,
              u_v, i_v, j_v, pi_v, pj_v):
    wid = lax.axis_index("s") * NC + lax.axis_index("c")
    base = wid * BPW
    pltpu.sync_copy(su_hbm.at[pl.ds(base * D, BPW * D)], u_v)
    pltpu.sync_copy(si_hbm.at[pl.ds(base * D, BPW * D)], i_v)
    pltpu.sync_copy(sj_hbm.at[pl.ds(base * D, BPW * D)], j_v)
    lane = lax.iota(jnp.int32, L)

    def blk_body(blk, carry):
        rbase = (blk * L + lane) * D
        acc_i = jnp.zeros((L,), jnp.float32)
        acc_j = jnp.zeros((L,), jnp.float32)
        for f in range(D):
            u = plsc.load_gather(u_v, [rbase + f])
            a = plsc.load_gather(i_v, [rbase + f])
            b = plsc.load_gather(j_v, [rbase + f])
            acc_i = acc_i + u * a
            acc_j = acc_j + u * b
        pi_v[pl.ds(blk * L, L)] = acc_i
        pj_v[pl.ds(blk * L, L)] = acc_j
        return carry

    lax.fori_loop(0, NBLK, blk_body, 0)
    pltpu.sync_copy(pi_v, out_i_hbm.at[pl.ds(base, BPW)])
    pltpu.sync_copy(pj_v, out_j_hbm.at[pl.ds(base, BPW)])


@jax.jit
def _run(user, item_i, item_j, embed_user, embed_item):
    mesh = plsc.VectorSubcoreMesh(core_axis_name="c", subcore_axis_name="s")
    params = pltpu.CompilerParams(
        needs_layout_passes=False, use_tc_tiling_on_sc=True)
    scan = pl.kernel(
        _scan_body,
        out_type=(
            jax.ShapeDtypeStruct((B * D,), jnp.float32),
            jax.ShapeDtypeStruct((B * D,), jnp.float32),
            jax.ShapeDtypeStruct((B * D,), jnp.float32),
        ),
        mesh=mesh,
        scratch_types=[
            pltpu.VMEM((B + L,), jnp.int32),
            pltpu.VMEM((B + L,), jnp.int32),
            pltpu.VMEM((B + L,), jnp.int32),
            pltpu.VMEM((NBUF, D, 128), jnp.float32),
            pltpu.VMEM((RB * D,), jnp.float32),
            pltpu.SemaphoreType.DMA,
            pltpu.SemaphoreType.DMA,
        ],
        compiler_params=params,
    )
    su, si, sj = scan(user, item_i, item_j, embed_user.T, embed_item.T)
    dot = pl.kernel(
        _dot_body,
        out_type=(
            jax.ShapeDtypeStruct((B,), jnp.float32),
            jax.ShapeDtypeStruct((B,), jnp.float32),
        ),
        mesh=mesh,
        scratch_types=[
            pltpu.VMEM((BPW * D,), jnp.float32),
            pltpu.VMEM((BPW * D,), jnp.float32),
            pltpu.VMEM((BPW * D,), jnp.float32),
            pltpu.VMEM((BPW,), jnp.float32),
            pltpu.VMEM((BPW,), jnp.float32),
        ],
        compiler_params=params,
    )
    return dot(su, si, sj)


def kernel(user, item_i, item_j, embed_user, embed_item):
    return _run(user.astype(jnp.int32), item_i.astype(jnp.int32),
                item_j.astype(jnp.int32), embed_user, embed_item)
